# baseline (speedup 1.0000x reference)
import jax
import jax.numpy as jnp
from jax import lax
from jax.experimental import pallas as pl
from jax.experimental.pallas import tpu as pltpu
from jax.experimental.pallas import tpu_sc as plsc

N = 4096
D = 128
E = 131072

NC = 2
NS = 16
L = 16

_DEG_ROWS = E // 128
_DEG_RPW = _DEG_ROWS // (NC * NS)


def _deg_body(dst2d, ew2d, degp, shared, idxb, valb, zb):
    c = lax.axis_index("c")
    s = lax.axis_index("s")
    w = s * NC + c
    for q in range(256 // L):
        zb[pl.ds(q * L, L)] = jnp.zeros((L,), jnp.float32)
    pltpu.sync_copy(zb, shared.at[pl.ds(s * 256, 256)])
    plsc.subcore_barrier()
    pltpu.sync_copy(dst2d.at[pl.ds(w * _DEG_RPW, _DEG_RPW)], idxb)
    pltpu.sync_copy(ew2d.at[pl.ds(w * _DEG_RPW, _DEG_RPW)], valb)

    def body(j, _):
        pltpu.sync_copy(valb.at[j], shared.at[idxb.at[j]], add=True)
        return 0

    lax.fori_loop(0, _DEG_RPW, body, 0)
    plsc.subcore_barrier()
    pltpu.sync_copy(shared.at[pl.ds(s * 256, 256)],
                    degp.at[c, pl.ds(s * 256, 256)])


def _sc_deg(dst2d, ew2d):
    mesh = plsc.VectorSubcoreMesh(core_axis_name="c", subcore_axis_name="s")
    f = pl.kernel(
        _deg_body,
        out_type=jax.ShapeDtypeStruct((NC, N), jnp.float32),
        mesh=mesh,
        scratch_types=[
            pltpu.VMEM_SHARED((N,), jnp.float32),
            pltpu.VMEM((_DEG_RPW, 128), jnp.int32),
            pltpu.VMEM((_DEG_RPW, 128), jnp.float32),
            pltpu.VMEM((256,), jnp.float32),
        ],
    )
    return f(dst2d, ew2d)


def _prep_body(degp_ref, dinv_ref):
    dinv_ref[...] = lax.rsqrt(degp_ref[0:1, :] + degp_ref[1:2, :] + 1.0)


def _tc_prep(degp):
    return pl.pallas_call(
        _prep_body,
        out_shape=jax.ShapeDtypeStruct((1, N), jnp.float32),
    )(degp)


_R = 256
_LIM = _R * N
_NCHUNK = N // _R
_EPW = E // NS
_SLICE = _LIM // NS


def _build_body(src_f, dst_f, ew_f, m_out,
                shared, srcb, dstb, ewb, fks, idxst, valst, zb,
                out_sem, clr_sem, sct_sem):
    c = lax.axis_index("c")
    s = lax.axis_index("s")
    lane = lax.iota(jnp.int32, L)

    def zloop(i, _):
        zb[pl.ds(i * L, L)] = jnp.zeros((L,), jnp.float32)
        return 0

    lax.fori_loop(0, _EPW // L, zloop, 0)

    e0 = s * _EPW
    pltpu.sync_copy(src_f.at[pl.ds(e0, _EPW)], srcb)
    pltpu.sync_copy(dst_f.at[pl.ds(e0, _EPW)], dstb)
    pltpu.sync_copy(ew_f.at[pl.ds(e0, _EPW)], ewb)

    def pre(i, _):
        sl = pl.ds(i * L, L)
        fks[sl] = srcb[sl] * N + dstb[sl]
        return 0

    lax.fori_loop(0, _EPW // L, pre, 0)

    out_desc = None
    for t in range(_NCHUNK // NC):
        lo_flat = (2 * t + c) * _LIM
        if out_desc is not None:
            out_desc.wait()
            out_desc = None
        clrs = []
        for q in range(_SLICE // _EPW):
            clrs.append(pltpu.async_copy(
                zb, shared.at[pl.ds(s * _SLICE + q * _EPW, _EPW)],
                clr_sem))

        def scan(jj, _):
            sac = (s * 65536 + jj * 1024 + lane) & (_LIM - 1)
            for kk in range(8):
                sl = pl.ds(jj * 128 + kk * L, L)
                rel = fks[sl] - lo_flat
                mask = (rel >= 0) & (rel < _LIM)
                idxst[sl] = jnp.where(mask, rel, sac)
                valst[sl] = jnp.where(mask, ewb[sl], 0.0)
            return 0

        lax.fori_loop(0, _EPW // 128, scan, 0)
        for d in clrs:
            d.wait()
        plsc.subcore_barrier()
        pltpu.async_copy(valst, shared.at[idxst], sct_sem, add=True).wait()
        plsc.subcore_barrier()
        out_desc = pltpu.async_copy(
            shared.at[pl.ds(s * _SLICE, _SLICE)],
            m_out.at[pl.ds(lo_flat + s * _SLICE, _SLICE)],
            out_sem)
    out_desc.wait()


def _sc_build(src_f, dst_f, ew_f):
    mesh = plsc.VectorSubcoreMesh(core_axis_name="c", subcore_axis_name="s")
    f = pl.kernel(
        _build_body,
        out_type=jax.ShapeDtypeStruct((N * N,), jnp.float32),
        mesh=mesh,
        scratch_types=[
            pltpu.VMEM_SHARED((_LIM,), jnp.float32),
            pltpu.VMEM((_EPW,), jnp.int32),
            pltpu.VMEM((_EPW,), jnp.int32),
            pltpu.VMEM((_EPW,), jnp.float32),
            pltpu.VMEM((_EPW,), jnp.int32),
            pltpu.VMEM((_EPW,), jnp.int32),
            pltpu.VMEM((_EPW,), jnp.float32),
            pltpu.VMEM((_EPW,), jnp.float32),
            pltpu.SemaphoreType.DMA,
            pltpu.SemaphoreType.DMA,
            pltpu.SemaphoreType.DMA,
        ],
    )
    return f(src_f, dst_f, ew_f)


_BM = 256


def _layer_body(m_ref, x_ref, w_ref, bias_ref, dinvf_ref,
                out_ref, u_ref, acc_ref):
    i = pl.program_id(0)

    @pl.when(i == 0)
    def _():
        xw = jnp.dot(x_ref[...], w_ref[...],
                     preferred_element_type=jnp.float32) + bias_ref[...]
        u = dinvf_ref[...] * xw
        u_ref[...] = u
        acc_ref[...] = u

    ub = u_ref[pl.ds(i * _BM, _BM), :]
    acc_ref[...] += lax.dot_general(
        m_ref[...], ub,
        dimension_numbers=(((0,), (0,)), ((), ())),
        preferred_element_type=jnp.float32)

    @pl.when(i == N // _BM - 1)
    def _():
        out_ref[...] = jnp.maximum(dinvf_ref[...] * acc_ref[...], 0.0)


def _tc_layer(m, x, w, bias2d, dinv_col):
    return pl.pallas_call(
        _layer_body,
        grid=(N // _BM,),
        in_specs=[
            pl.BlockSpec((_BM, N), lambda i: (i, 0)),
            pl.BlockSpec((N, D), lambda i: (0, 0)),
            pl.BlockSpec((D, D), lambda i: (0, 0)),
            pl.BlockSpec((1, D), lambda i: (0, 0)),
            pl.BlockSpec((N, 1), lambda i: (0, 0)),
        ],
        out_specs=pl.BlockSpec((N, D), lambda i: (0, 0)),
        out_shape=jax.ShapeDtypeStruct((N, D), jnp.float32),
        scratch_shapes=[pltpu.VMEM((N, D), jnp.float32),
                        pltpu.VMEM((N, D), jnp.float32)],
    )(m, x, w, bias2d, dinv_col)


def _dec_body(z_ref, zt_ref, adj_ref, x_ref, p_ref, enc_ref):
    zz = jnp.dot(z_ref[...], zt_ref[...], preferred_element_type=jnp.float32)
    p_ref[...] = jax.nn.sigmoid(zz + adj_ref[...])
    enc_ref[...] = z_ref[...] + x_ref[...]


def _tc_dec(z, zt, adj, x):
    return pl.pallas_call(
        _dec_body,
        grid=(N // _BM,),
        in_specs=[
            pl.BlockSpec((_BM, D), lambda i: (i, 0)),
            pl.BlockSpec((D, N), lambda i: (0, 0)),
            pl.BlockSpec((_BM, N), lambda i: (i, 0)),
            pl.BlockSpec((_BM, D), lambda i: (i, 0)),
        ],
        out_specs=[
            pl.BlockSpec((_BM, N), lambda i: (i, 0)),
            pl.BlockSpec((_BM, D), lambda i: (i, 0)),
        ],
        out_shape=[
            jax.ShapeDtypeStruct((N, N), jnp.float32),
            jax.ShapeDtypeStruct((N, D), jnp.float32),
        ],
    )(z, zt, adj, x)


def kernel(node_features, edge_list, edge_attr, batch, W1, b1, W2, b2):
    src_f = edge_list[0]
    dst_f = edge_list[1]
    dst2d = dst_f.reshape(_DEG_ROWS, 128)
    ew2d = edge_attr.reshape(_DEG_ROWS, 128)

    degp = _sc_deg(dst2d, ew2d)
    dinv_col = _tc_prep(degp).reshape(N, 1)
    m_flat = _sc_build(src_f, dst_f, edge_attr)
    m_mat = m_flat.reshape(N, N)

    h = _tc_layer(m_mat, node_features, W1, b1.reshape(1, D), dinv_col)
    z = _tc_layer(m_mat, h, W2, b2.reshape(1, D), dinv_col)
    p, enc = _tc_dec(z, z.T, m_mat, node_features)
    return (enc, edge_list, p)


# speedup vs baseline: 1.1833x; 1.1833x over previous
import jax
import jax.numpy as jnp
from jax import lax
from jax.experimental import pallas as pl
from jax.experimental.pallas import tpu as pltpu
from jax.experimental.pallas import tpu_sc as plsc

N = 4096
D = 128
E = 131072

NC = 2
NS = 16
L = 16

_DEG_ROWS = E // 128
_DEG_RPW = _DEG_ROWS // (NC * NS)


def _deg_body(dst2d, ew2d, degp, shared, idxb, valb, zb):
    c = lax.axis_index("c")
    s = lax.axis_index("s")
    w = s * NC + c
    for q in range(256 // L):
        zb[pl.ds(q * L, L)] = jnp.zeros((L,), jnp.float32)
    pltpu.sync_copy(zb, shared.at[pl.ds(s * 256, 256)])
    plsc.subcore_barrier()
    pltpu.sync_copy(dst2d.at[pl.ds(w * _DEG_RPW, _DEG_RPW)], idxb)
    pltpu.sync_copy(ew2d.at[pl.ds(w * _DEG_RPW, _DEG_RPW)], valb)

    def body(j, _):
        pltpu.sync_copy(valb.at[j], shared.at[idxb.at[j]], add=True)
        return 0

    lax.fori_loop(0, _DEG_RPW, body, 0)
    plsc.subcore_barrier()
    pltpu.sync_copy(shared.at[pl.ds(s * 256, 256)],
                    degp.at[c, pl.ds(s * 256, 256)])


def _sc_deg(dst2d, ew2d):
    mesh = plsc.VectorSubcoreMesh(core_axis_name="c", subcore_axis_name="s")
    f = pl.kernel(
        _deg_body,
        out_type=jax.ShapeDtypeStruct((NC, N), jnp.float32),
        mesh=mesh,
        scratch_types=[
            pltpu.VMEM_SHARED((N,), jnp.float32),
            pltpu.VMEM((_DEG_RPW, 128), jnp.int32),
            pltpu.VMEM((_DEG_RPW, 128), jnp.float32),
            pltpu.VMEM((256,), jnp.float32),
        ],
    )
    return f(dst2d, ew2d)


def _prep_body(degp_ref, dinv_ref):
    dinv_ref[...] = lax.rsqrt(degp_ref[0:1, :] + degp_ref[1:2, :] + 1.0)


def _tc_prep(degp):
    return pl.pallas_call(
        _prep_body,
        out_shape=jax.ShapeDtypeStruct((1, N), jnp.float32),
    )(degp)


_R = 256
_LIM = _R * N
_NCHUNK = N // _R
_EPW = E // NS
_SLICE = _LIM // NS


def _build_body(src_f, dst_f, ew_f, m_out,
                shared, srcb, dstb, ewb, fks, idxst, valst, zb,
                out_sem, clr_sem, sct_sem):
    c = lax.axis_index("c")
    s = lax.axis_index("s")
    lane = lax.iota(jnp.int32, L)

    def zloop(i, _):
        zb[pl.ds(i * L, L)] = jnp.zeros((L,), jnp.float32)
        return 0

    lax.fori_loop(0, _EPW // L, zloop, 0)

    e0 = s * _EPW
    pltpu.sync_copy(src_f.at[pl.ds(e0, _EPW)], srcb)
    pltpu.sync_copy(dst_f.at[pl.ds(e0, _EPW)], dstb)
    pltpu.sync_copy(ew_f.at[pl.ds(e0, _EPW)], ewb)

    def pre(i, _):
        sl = pl.ds(i * L, L)
        fks[sl] = srcb[sl] * N + dstb[sl]
        return 0

    lax.fori_loop(0, _EPW // L, pre, 0)

    out_descs = []
    for t in range(_NCHUNK // NC):
        lo_flat = (2 * t + c) * _LIM
        for d in out_descs:
            d.wait()
        out_descs = []
        clrs = []
        for q in range(_SLICE // _EPW):
            clrs.append(pltpu.async_copy(
                zb, shared.at[pl.ds(s * _SLICE + q * _EPW, _EPW)],
                clr_sem))

        def scan(jj, _):
            sac = (s * 65536 + jj * 1024 + lane) & (_LIM - 1)
            for kk in range(8):
                sl = pl.ds(jj * 128 + kk * L, L)
                rel = fks[sl] - lo_flat
                mask = (rel >= 0) & (rel < _LIM)
                idxst[sl] = jnp.where(mask, rel, sac)
                valst[sl] = jnp.where(mask, ewb[sl], 0.0)
            return 0

        lax.fori_loop(0, _EPW // 128, scan, 0)
        for d in clrs:
            d.wait()
        plsc.subcore_barrier()
        pltpu.async_copy(valst, shared.at[idxst], sct_sem, add=True).wait()
        plsc.subcore_barrier()
        row0 = (2 * t + c) * _R + s * (_R // NS)
        for j in range(_R // NS):
            out_descs.append(pltpu.async_copy(
                shared.at[pl.ds(s * _SLICE + j * N, N)],
                m_out.at[row0 + j], out_sem))
    for d in out_descs:
        d.wait()


def _sc_build(src_f, dst_f, ew_f):
    mesh = plsc.VectorSubcoreMesh(core_axis_name="c", subcore_axis_name="s")
    f = pl.kernel(
        _build_body,
        out_type=jax.ShapeDtypeStruct((N, N), jnp.float32),
        mesh=mesh,
        scratch_types=[
            pltpu.VMEM_SHARED((_LIM,), jnp.float32),
            pltpu.VMEM((_EPW,), jnp.int32),
            pltpu.VMEM((_EPW,), jnp.int32),
            pltpu.VMEM((_EPW,), jnp.float32),
            pltpu.VMEM((_EPW,), jnp.int32),
            pltpu.VMEM((_EPW,), jnp.int32),
            pltpu.VMEM((_EPW,), jnp.float32),
            pltpu.VMEM((_EPW,), jnp.float32),
            pltpu.SemaphoreType.DMA,
            pltpu.SemaphoreType.DMA,
            pltpu.SemaphoreType.DMA,
        ],
    )
    return f(src_f, dst_f, ew_f)


_BM = 256


def _layer_body(m_ref, x_ref, w_ref, bias_ref, dinvf_ref,
                out_ref, u_ref, acc_ref):
    i = pl.program_id(0)

    @pl.when(i == 0)
    def _():
        xw = jnp.dot(x_ref[...], w_ref[...],
                     preferred_element_type=jnp.float32) + bias_ref[...]
        u = dinvf_ref[...] * xw
        u_ref[...] = u
        acc_ref[...] = u

    ub = u_ref[pl.ds(i * _BM, _BM), :]
    acc_ref[...] += lax.dot_general(
        m_ref[...], ub,
        dimension_numbers=(((0,), (0,)), ((), ())),
        preferred_element_type=jnp.float32)

    @pl.when(i == N // _BM - 1)
    def _():
        out_ref[...] = jnp.maximum(dinvf_ref[...] * acc_ref[...], 0.0)


def _tc_layer(m, x, w, bias2d, dinv_col):
    return pl.pallas_call(
        _layer_body,
        grid=(N // _BM,),
        in_specs=[
            pl.BlockSpec((_BM, N), lambda i: (i, 0)),
            pl.BlockSpec((N, D), lambda i: (0, 0)),
            pl.BlockSpec((D, D), lambda i: (0, 0)),
            pl.BlockSpec((1, D), lambda i: (0, 0)),
            pl.BlockSpec((N, 1), lambda i: (0, 0)),
        ],
        out_specs=pl.BlockSpec((N, D), lambda i: (0, 0)),
        out_shape=jax.ShapeDtypeStruct((N, D), jnp.float32),
        scratch_shapes=[pltpu.VMEM((N, D), jnp.float32),
                        pltpu.VMEM((N, D), jnp.float32)],
    )(m, x, w, bias2d, dinv_col)


def _dec_body(z_ref, zt_ref, adj_ref, x_ref, p_ref, enc_ref):
    zz = jnp.dot(z_ref[...], zt_ref[...], preferred_element_type=jnp.float32)
    p_ref[...] = jax.nn.sigmoid(zz + adj_ref[...])
    enc_ref[...] = z_ref[...] + x_ref[...]


def _tc_dec(z, zt, adj, x):
    return pl.pallas_call(
        _dec_body,
        grid=(N // _BM,),
        in_specs=[
            pl.BlockSpec((_BM, D), lambda i: (i, 0)),
            pl.BlockSpec((D, N), lambda i: (0, 0)),
            pl.BlockSpec((_BM, N), lambda i: (i, 0)),
            pl.BlockSpec((_BM, D), lambda i: (i, 0)),
        ],
        out_specs=[
            pl.BlockSpec((_BM, N), lambda i: (i, 0)),
            pl.BlockSpec((_BM, D), lambda i: (i, 0)),
        ],
        out_shape=[
            jax.ShapeDtypeStruct((N, N), jnp.float32),
            jax.ShapeDtypeStruct((N, D), jnp.float32),
        ],
    )(z, zt, adj, x)


def kernel(node_features, edge_list, edge_attr, batch, W1, b1, W2, b2):
    src_f = edge_list[0]
    dst_f = edge_list[1]
    dst2d = dst_f.reshape(_DEG_ROWS, 128)
    ew2d = edge_attr.reshape(_DEG_ROWS, 128)

    degp = _sc_deg(dst2d, ew2d)
    dinv_col = _tc_prep(degp).reshape(N, 1)
    m_mat = _sc_build(src_f, dst_f, edge_attr)

    h = _tc_layer(m_mat, node_features, W1, b1.reshape(1, D), dinv_col)
    z = _tc_layer(m_mat, h, W2, b2.reshape(1, D), dinv_col)
    p, enc = _tc_dec(z, z.T, m_mat, node_features)
    return (enc, edge_list, p)


# speedup vs baseline: 1.8028x; 1.5236x over previous
import jax
import jax.numpy as jnp
from jax import lax
from jax.experimental import pallas as pl
from jax.experimental.pallas import tpu as pltpu
from jax.experimental.pallas import tpu_sc as plsc

N = 4096
D = 128
E = 131072

NC = 2
NS = 16
L = 16

_DEG_ROWS = E // 128
_DEG_RPW = _DEG_ROWS // (NC * NS)


def _deg_body(dst2d, ew2d, degp, shared, idxb, valb, zb):
    c = lax.axis_index("c")
    s = lax.axis_index("s")
    w = s * NC + c
    for q in range(256 // L):
        zb[pl.ds(q * L, L)] = jnp.zeros((L,), jnp.float32)
    pltpu.sync_copy(zb, shared.at[pl.ds(s * 256, 256)])
    plsc.subcore_barrier()
    pltpu.sync_copy(dst2d.at[pl.ds(w * _DEG_RPW, _DEG_RPW)], idxb)
    pltpu.sync_copy(ew2d.at[pl.ds(w * _DEG_RPW, _DEG_RPW)], valb)

    def body(j, _):
        pltpu.sync_copy(valb.at[j], shared.at[idxb.at[j]], add=True)
        return 0

    lax.fori_loop(0, _DEG_RPW, body, 0)
    plsc.subcore_barrier()
    pltpu.sync_copy(shared.at[pl.ds(s * 256, 256)],
                    degp.at[c, pl.ds(s * 256, 256)])


def _sc_deg(dst2d, ew2d):
    mesh = plsc.VectorSubcoreMesh(core_axis_name="c", subcore_axis_name="s")
    f = pl.kernel(
        _deg_body,
        out_type=jax.ShapeDtypeStruct((NC, N), jnp.float32),
        mesh=mesh,
        scratch_types=[
            pltpu.VMEM_SHARED((N,), jnp.float32),
            pltpu.VMEM((_DEG_RPW, 128), jnp.int32),
            pltpu.VMEM((_DEG_RPW, 128), jnp.float32),
            pltpu.VMEM((256,), jnp.float32),
        ],
    )
    return f(dst2d, ew2d)


def _prep_body(degp_ref, dinv_ref):
    dinv_ref[...] = lax.rsqrt(degp_ref[0:1, :] + degp_ref[1:2, :] + 1.0)


def _tc_prep(degp):
    return pl.pallas_call(
        _prep_body,
        out_shape=jax.ShapeDtypeStruct((1, N), jnp.float32),
    )(degp)


_R = 256
_LIM = _R * N
_NCHUNK = N // _R
_EPW = E // NS
_SLICE = _LIM // NS
_NB = _NCHUNK // NC
_CAP = 768
_CAPA = 1024


def _build_body(src_f, dst_f, ew_f, m_out,
                shared, srcb, dstb, ewb, fks, idxst, valst, zb,
                idxs, curv,
                out_sem, clr_sem, sct_sem):
    fkbuf = dstb
    vvbuf = valst
    c = lax.axis_index("c")
    s = lax.axis_index("s")
    lane = lax.iota(jnp.int32, L)

    def zloop(i, _):
        zb[pl.ds(i * L, L)] = jnp.zeros((L,), jnp.float32)
        return 0

    lax.fori_loop(0, _EPW // L, zloop, 0)

    e0 = s * _EPW
    pltpu.sync_copy(src_f.at[pl.ds(e0, _EPW)], srcb)
    pltpu.sync_copy(dst_f.at[pl.ds(e0, _EPW)], dstb)
    pltpu.sync_copy(ew_f.at[pl.ds(e0, _EPW)], ewb)

    def pre(i, _):
        sl = pl.ds(i * L, L)
        fks[sl] = srcb[sl] * N + dstb[sl]
        return 0

    lax.fori_loop(0, _EPW // L, pre, 0)

    def pfill(q, _):
        for b in range(_NB):
            base = (2 * b + c) * _LIM
            fkbuf[pl.ds(b * _CAPA + q * L, L)] = base + q * L + lane
            vvbuf[pl.ds(b * _CAPA + q * L, L)] = jnp.zeros((L,), jnp.float32)
        return 0

    lax.fori_loop(0, _CAP // L, pfill, 0)

    def bstep(i, cur):
        sl = pl.ds(i * L, L)
        fk = fks[sl]
        ev = ewb[sl]
        cid = lax.shift_right_logical(fk, 20)
        valid = (cid & 1) == c
        bof = lax.shift_right_logical(cid, 1)
        nxt = []
        for b in range(_NB):
            mask = (bof == b) & valid
            pos = plsc.cumsum(jnp.where(mask, 1, 0)) - 1
            idx = b * _CAPA + cur[b] + pos
            idx = jnp.where(mask & (idx < b * _CAPA + _CAP), idx,
                            b * _CAPA + _CAP + lane)
            plsc.store_scatter(fkbuf, [idx], fk)
            plsc.store_scatter(vvbuf, [idx], ev)
            nxt.append(cur[b] + plsc.all_reduce_population_count(mask))
        return tuple(nxt)

    fin = lax.fori_loop(0, _EPW // L, bstep,
                        tuple(jnp.zeros((L,), jnp.int32) for _ in range(_NB)))
    mx = fin[0]
    for b in range(1, _NB):
        mx = jnp.maximum(mx, fin[b])
    curv[pl.ds(0, L)] = mx

    out_descs = []
    for t in range(_NCHUNK // NC):
        lo_flat = (2 * t + c) * _LIM
        for d in out_descs:
            d.wait()
        out_descs = []
        clrs = []
        for q in range(_SLICE // _EPW):
            clrs.append(pltpu.async_copy(
                zb, shared.at[pl.ds(s * _SLICE + q * _EPW, _EPW)],
                clr_sem))

        ovf = curv[pl.ds(0, L)][0] > _CAP

        @pl.when(jnp.logical_not(ovf))
        def _():
            def bprep(q, _):
                sl = pl.ds(q * L, L)
                idxs[sl] = fkbuf[pl.ds(t * _CAPA + q * L, L)] - lo_flat
                return 0

            lax.fori_loop(0, _CAP // L, bprep, 0)

        @pl.when(ovf)
        def _():
            def scan(jj, _):
                sac = (s * 65536 + jj * 1024 + lane) & (_LIM - 1)
                for kk in range(8):
                    sl = pl.ds(jj * 128 + kk * L, L)
                    rel = fks[sl] - lo_flat
                    mask = (rel >= 0) & (rel < _LIM)
                    idxst[sl] = jnp.where(mask, rel, sac)
                    valst[sl] = jnp.where(mask, ewb[sl], 0.0)
                return 0

            lax.fori_loop(0, _EPW // 128, scan, 0)

        for d in clrs:
            d.wait()
        plsc.subcore_barrier()

        @pl.when(jnp.logical_not(ovf))
        def _():
            pltpu.async_copy(valst.at[pl.ds(t * _CAPA, _CAP)],
                             shared.at[idxs], sct_sem, add=True).wait()

        @pl.when(ovf)
        def _():
            pltpu.async_copy(valst, shared.at[idxst], sct_sem,
                             add=True).wait()

        plsc.subcore_barrier()
        row0 = (2 * t + c) * _R + s * (_R // NS)
        for j in range(_R // NS):
            out_descs.append(pltpu.async_copy(
                shared.at[pl.ds(s * _SLICE + j * N, N)],
                m_out.at[row0 + j], out_sem))
    for d in out_descs:
        d.wait()


def _sc_build(src_f, dst_f, ew_f):
    mesh = plsc.VectorSubcoreMesh(core_axis_name="c", subcore_axis_name="s")
    f = pl.kernel(
        _build_body,
        out_type=jax.ShapeDtypeStruct((N, N), jnp.float32),
        mesh=mesh,
        scratch_types=[
            pltpu.VMEM_SHARED((_LIM,), jnp.float32),
            pltpu.VMEM((_EPW,), jnp.int32),
            pltpu.VMEM((_EPW,), jnp.int32),
            pltpu.VMEM((_EPW,), jnp.float32),
            pltpu.VMEM((_EPW,), jnp.int32),
            pltpu.VMEM((_EPW,), jnp.int32),
            pltpu.VMEM((_EPW,), jnp.float32),
            pltpu.VMEM((_EPW,), jnp.float32),
            pltpu.VMEM((_CAP,), jnp.int32),
            pltpu.VMEM((L,), jnp.int32),
            pltpu.SemaphoreType.DMA,
            pltpu.SemaphoreType.DMA,
            pltpu.SemaphoreType.DMA,
        ],
        compiler_params=pltpu.CompilerParams(needs_layout_passes=False),
    )
    return f(src_f, dst_f, ew_f)


_BM = 256


def _layer_body(m_ref, x_ref, w_ref, bias_ref, dinvf_ref,
                out_ref, u_ref, acc_ref):
    i = pl.program_id(0)

    @pl.when(i == 0)
    def _():
        xw = jnp.dot(x_ref[...], w_ref[...],
                     preferred_element_type=jnp.float32) + bias_ref[...]
        u = dinvf_ref[...] * xw
        u_ref[...] = u
        acc_ref[...] = u

    ub = u_ref[pl.ds(i * _BM, _BM), :]
    acc_ref[...] += lax.dot_general(
        m_ref[...], ub,
        dimension_numbers=(((0,), (0,)), ((), ())),
        preferred_element_type=jnp.float32)

    @pl.when(i == N // _BM - 1)
    def _():
        out_ref[...] = jnp.maximum(dinvf_ref[...] * acc_ref[...], 0.0)


def _tc_layer(m, x, w, bias2d, dinv_col):
    return pl.pallas_call(
        _layer_body,
        grid=(N // _BM,),
        in_specs=[
            pl.BlockSpec((_BM, N), lambda i: (i, 0)),
            pl.BlockSpec((N, D), lambda i: (0, 0)),
            pl.BlockSpec((D, D), lambda i: (0, 0)),
            pl.BlockSpec((1, D), lambda i: (0, 0)),
            pl.BlockSpec((N, 1), lambda i: (0, 0)),
        ],
        out_specs=pl.BlockSpec((N, D), lambda i: (0, 0)),
        out_shape=jax.ShapeDtypeStruct((N, D), jnp.float32),
        scratch_shapes=[pltpu.VMEM((N, D), jnp.float32),
                        pltpu.VMEM((N, D), jnp.float32)],
    )(m, x, w, bias2d, dinv_col)


def _dec_body(z_ref, zt_ref, adj_ref, x_ref, p_ref, enc_ref):
    zz = jnp.dot(z_ref[...], zt_ref[...], preferred_element_type=jnp.float32)
    p_ref[...] = jax.nn.sigmoid(zz + adj_ref[...])
    enc_ref[...] = z_ref[...] + x_ref[...]


def _tc_dec(z, zt, adj, x):
    return pl.pallas_call(
        _dec_body,
        grid=(N // _BM,),
        in_specs=[
            pl.BlockSpec((_BM, D), lambda i: (i, 0)),
            pl.BlockSpec((D, N), lambda i: (0, 0)),
            pl.BlockSpec((_BM, N), lambda i: (i, 0)),
            pl.BlockSpec((_BM, D), lambda i: (i, 0)),
        ],
        out_specs=[
            pl.BlockSpec((_BM, N), lambda i: (i, 0)),
            pl.BlockSpec((_BM, D), lambda i: (i, 0)),
        ],
        out_shape=[
            jax.ShapeDtypeStruct((N, N), jnp.float32),
            jax.ShapeDtypeStruct((N, D), jnp.float32),
        ],
    )(z, zt, adj, x)


def kernel(node_features, edge_list, edge_attr, batch, W1, b1, W2, b2):
    src_f = edge_list[0]
    dst_f = edge_list[1]
    dst2d = dst_f.reshape(_DEG_ROWS, 128)
    ew2d = edge_attr.reshape(_DEG_ROWS, 128)

    degp = _sc_deg(dst2d, ew2d)
    dinv_col = _tc_prep(degp).reshape(N, 1)
    m_mat = _sc_build(src_f, dst_f, edge_attr)

    h = _tc_layer(m_mat, node_features, W1, b1.reshape(1, D), dinv_col)
    z = _tc_layer(m_mat, h, W2, b2.reshape(1, D), dinv_col)
    p, enc = _tc_dec(z, z.T, m_mat, node_features)
    return (enc, edge_list, p)


# speedup vs baseline: 1.8668x; 1.0355x over previous
import jax
import jax.numpy as jnp
from jax import lax
from jax.experimental import pallas as pl
from jax.experimental.pallas import tpu as pltpu
from jax.experimental.pallas import tpu_sc as plsc

N = 4096
D = 128
E = 131072

NC = 2
NS = 16
L = 16

_DEG_ROWS = E // 128
_DEG_RPW = _DEG_ROWS // (NC * NS)


def _deg_body(dst2d, ew2d, degp, shared, idxb, valb, zb):
    c = lax.axis_index("c")
    s = lax.axis_index("s")
    w = s * NC + c
    for q in range(256 // L):
        zb[pl.ds(q * L, L)] = jnp.zeros((L,), jnp.float32)
    pltpu.sync_copy(zb, shared.at[pl.ds(s * 256, 256)])
    plsc.subcore_barrier()
    pltpu.sync_copy(dst2d.at[pl.ds(w * _DEG_RPW, _DEG_RPW)], idxb)
    pltpu.sync_copy(ew2d.at[pl.ds(w * _DEG_RPW, _DEG_RPW)], valb)

    def body(j, _):
        pltpu.sync_copy(valb.at[j], shared.at[idxb.at[j]], add=True)
        return 0

    lax.fori_loop(0, _DEG_RPW, body, 0)
    plsc.subcore_barrier()
    pltpu.sync_copy(shared.at[pl.ds(s * 256, 256)],
                    degp.at[c, pl.ds(s * 256, 256)])


def _sc_deg(dst2d, ew2d):
    mesh = plsc.VectorSubcoreMesh(core_axis_name="c", subcore_axis_name="s")
    f = pl.kernel(
        _deg_body,
        out_type=jax.ShapeDtypeStruct((NC, N), jnp.float32),
        mesh=mesh,
        scratch_types=[
            pltpu.VMEM_SHARED((N,), jnp.float32),
            pltpu.VMEM((_DEG_RPW, 128), jnp.int32),
            pltpu.VMEM((_DEG_RPW, 128), jnp.float32),
            pltpu.VMEM((256,), jnp.float32),
        ],
    )
    return f(dst2d, ew2d)


def _prep_body(degp_ref, dinv_ref):
    dinv_ref[...] = lax.rsqrt(degp_ref[0:1, :] + degp_ref[1:2, :] + 1.0)


def _tc_prep(degp):
    return pl.pallas_call(
        _prep_body,
        out_shape=jax.ShapeDtypeStruct((1, N), jnp.float32),
    )(degp)


_R = 256
_LIM = _R * N
_NCHUNK = N // _R
_EPW = E // NS
_SLICE = _LIM // NS
_NB = _NCHUNK // NC
_CAP = 768
_CAPA = 1024


def _build_body(src_f, dst_f, ew_f, m_out,
                shared, srcb, dstb, ewb, fks, idxst, valst, zb,
                idxs, curv,
                out_sem, clr_sem, sct_sem):
    fkbuf = dstb
    vvbuf = valst
    c = lax.axis_index("c")
    s = lax.axis_index("s")
    lane = lax.iota(jnp.int32, L)

    def zloop(i, _):
        zb[pl.ds(i * L, L)] = jnp.zeros((L,), jnp.float32)
        return 0

    lax.fori_loop(0, _EPW // L, zloop, 0)

    e0 = s * _EPW
    pltpu.sync_copy(src_f.at[pl.ds(e0, _EPW)], srcb)
    pltpu.sync_copy(dst_f.at[pl.ds(e0, _EPW)], dstb)
    pltpu.sync_copy(ew_f.at[pl.ds(e0, _EPW)], ewb)

    def pre(i, _):
        sl = pl.ds(i * L, L)
        fks[sl] = srcb[sl] * N + dstb[sl]
        return 0

    lax.fori_loop(0, _EPW // L, pre, 0)

    def pfill(q, _):
        for b in range(_NB):
            base = (2 * b + c) * _LIM
            fkbuf[pl.ds(b * _CAPA + q * L, L)] = base + q * L + lane
            vvbuf[pl.ds(b * _CAPA + q * L, L)] = jnp.zeros((L,), jnp.float32)
        return 0

    lax.fori_loop(0, _CAP // L, pfill, 0)

    def bstep(i, cur):
        sl = pl.ds(i * L, L)
        fk = fks[sl]
        ev = ewb[sl]
        cid = lax.shift_right_logical(fk, 20)
        valid = (cid & 1) == c
        bof = lax.shift_right_logical(cid, 1)
        nxt = []
        for b in range(_NB):
            mask = (bof == b) & valid
            pos = plsc.cumsum(jnp.where(mask, 1, 0)) - 1
            idx = b * _CAPA + cur[b] + pos
            idx = jnp.where(mask & (idx < b * _CAPA + _CAP), idx,
                            b * _CAPA + _CAP + lane)
            plsc.store_scatter(fkbuf, [idx], fk)
            plsc.store_scatter(vvbuf, [idx], ev)
            nxt.append(cur[b] + plsc.all_reduce_population_count(mask))
        return tuple(nxt)

    fin = lax.fori_loop(0, _EPW // L, bstep,
                        tuple(jnp.zeros((L,), jnp.int32) for _ in range(_NB)))
    mx = fin[0]
    for b in range(1, _NB):
        mx = jnp.maximum(mx, fin[b])
    curv[pl.ds(0, L)] = mx

    out_descs = []
    for t in range(_NCHUNK // NC):
        lo_flat = (2 * t + c) * _LIM
        for d in out_descs:
            d.wait()
        out_descs = []
        clrs = []
        for q in range(_SLICE // _EPW):
            clrs.append(pltpu.async_copy(
                zb, shared.at[pl.ds(s * _SLICE + q * _EPW, _EPW)],
                clr_sem))

        ovf = curv[pl.ds(0, L)][0] > _CAP

        @pl.when(jnp.logical_not(ovf))
        def _():
            def bprep(q, _):
                sl = pl.ds(q * L, L)
                idxs[sl] = fkbuf[pl.ds(t * _CAPA + q * L, L)] - lo_flat
                return 0

            lax.fori_loop(0, _CAP // L, bprep, 0)

        @pl.when(ovf)
        def _():
            def scan(jj, _):
                sac = (s * 65536 + jj * 1024 + lane) & (_LIM - 1)
                for kk in range(8):
                    sl = pl.ds(jj * 128 + kk * L, L)
                    rel = fks[sl] - lo_flat
                    mask = (rel >= 0) & (rel < _LIM)
                    idxst[sl] = jnp.where(mask, rel, sac)
                    valst[sl] = jnp.where(mask, ewb[sl], 0.0)
                return 0

            lax.fori_loop(0, _EPW // 128, scan, 0)

        for d in clrs:
            d.wait()
        plsc.subcore_barrier()

        @pl.when(jnp.logical_not(ovf))
        def _():
            pltpu.async_copy(valst.at[pl.ds(t * _CAPA, _CAP)],
                             shared.at[idxs], sct_sem, add=True).wait()

        @pl.when(ovf)
        def _():
            pltpu.async_copy(valst, shared.at[idxst], sct_sem,
                             add=True).wait()

        plsc.subcore_barrier()
        row0 = (2 * t + c) * _R + s * (_R // NS)
        for j in range(_R // NS):
            out_descs.append(pltpu.async_copy(
                shared.at[pl.ds(s * _SLICE + j * N, N)],
                m_out.at[row0 + j], out_sem))
    for d in out_descs:
        d.wait()


def _sc_build(src_f, dst_f, ew_f):
    mesh = plsc.VectorSubcoreMesh(core_axis_name="c", subcore_axis_name="s")
    f = pl.kernel(
        _build_body,
        out_type=jax.ShapeDtypeStruct((N, N), jnp.float32),
        mesh=mesh,
        scratch_types=[
            pltpu.VMEM_SHARED((_LIM,), jnp.float32),
            pltpu.VMEM((_EPW,), jnp.int32),
            pltpu.VMEM((_EPW,), jnp.int32),
            pltpu.VMEM((_EPW,), jnp.float32),
            pltpu.VMEM((_EPW,), jnp.int32),
            pltpu.VMEM((_EPW,), jnp.int32),
            pltpu.VMEM((_EPW,), jnp.float32),
            pltpu.VMEM((_EPW,), jnp.float32),
            pltpu.VMEM((_CAP,), jnp.int32),
            pltpu.VMEM((L,), jnp.int32),
            pltpu.SemaphoreType.DMA,
            pltpu.SemaphoreType.DMA,
            pltpu.SemaphoreType.DMA,
        ],
        compiler_params=pltpu.CompilerParams(needs_layout_passes=False),
    )
    return f(src_f, dst_f, ew_f)


_BM = 512


def _layer_body(m_ref, x_ref, w_ref, bias_ref, dinvf_ref,
                out_ref, u_ref, acc_ref):
    i = pl.program_id(0)

    @pl.when(i == 0)
    def _():
        xw = jnp.dot(x_ref[...], w_ref[...],
                     preferred_element_type=jnp.float32) + bias_ref[...]
        u = dinvf_ref[...] * xw
        u_ref[...] = u
        acc_ref[...] = u

    ub = u_ref[pl.ds(i * _BM, _BM), :]
    acc_ref[...] += lax.dot_general(
        m_ref[...], ub,
        dimension_numbers=(((0,), (0,)), ((), ())),
        preferred_element_type=jnp.float32)

    @pl.when(i == N // _BM - 1)
    def _():
        out_ref[...] = jnp.maximum(dinvf_ref[...] * acc_ref[...], 0.0)


def _tc_layer(m, x, w, bias2d, dinv_col):
    return pl.pallas_call(
        _layer_body,
        grid=(N // _BM,),
        in_specs=[
            pl.BlockSpec((_BM, N), lambda i: (i, 0)),
            pl.BlockSpec((N, D), lambda i: (0, 0)),
            pl.BlockSpec((D, D), lambda i: (0, 0)),
            pl.BlockSpec((1, D), lambda i: (0, 0)),
            pl.BlockSpec((N, 1), lambda i: (0, 0)),
        ],
        out_specs=pl.BlockSpec((N, D), lambda i: (0, 0)),
        out_shape=jax.ShapeDtypeStruct((N, D), jnp.float32),
        scratch_shapes=[pltpu.VMEM((N, D), jnp.float32),
                        pltpu.VMEM((N, D), jnp.float32)],
    )(m, x, w, bias2d, dinv_col)


def _dec_body(z_ref, zt_ref, adj_ref, x_ref, p_ref, enc_ref):
    zz = jnp.dot(z_ref[...], zt_ref[...], preferred_element_type=jnp.float32)
    p_ref[...] = jax.nn.sigmoid(zz + adj_ref[...])
    enc_ref[...] = z_ref[...] + x_ref[...]


def _tc_dec(z, zt, adj, x):
    return pl.pallas_call(
        _dec_body,
        grid=(N // _BM,),
        in_specs=[
            pl.BlockSpec((_BM, D), lambda i: (i, 0)),
            pl.BlockSpec((D, N), lambda i: (0, 0)),
            pl.BlockSpec((_BM, N), lambda i: (i, 0)),
            pl.BlockSpec((_BM, D), lambda i: (i, 0)),
        ],
        out_specs=[
            pl.BlockSpec((_BM, N), lambda i: (i, 0)),
            pl.BlockSpec((_BM, D), lambda i: (i, 0)),
        ],
        out_shape=[
            jax.ShapeDtypeStruct((N, N), jnp.float32),
            jax.ShapeDtypeStruct((N, D), jnp.float32),
        ],
    )(z, zt, adj, x)


def kernel(node_features, edge_list, edge_attr, batch, W1, b1, W2, b2):
    src_f = edge_list[0]
    dst_f = edge_list[1]
    dst2d = dst_f.reshape(_DEG_ROWS, 128)
    ew2d = edge_attr.reshape(_DEG_ROWS, 128)

    degp = _sc_deg(dst2d, ew2d)
    dinv_col = _tc_prep(degp).reshape(N, 1)
    m_mat = _sc_build(src_f, dst_f, edge_attr)

    h = _tc_layer(m_mat, node_features, W1, b1.reshape(1, D), dinv_col)
    z = _tc_layer(m_mat, h, W2, b2.reshape(1, D), dinv_col)
    p, enc = _tc_dec(z, z.T, m_mat, node_features)
    return (enc, edge_list, p)
